# baseline (speedup 1.0000x reference)
import functools

import jax
import jax.numpy as jnp
from jax import lax
from jax.experimental import pallas as pl
from jax.experimental.pallas import tpu as pltpu
from jax.experimental.pallas import tpu_sc as plsc

N = 100000
E = 1600000
NIN = 512
NOUT = 512
LO = N - NOUT

_info = plsc.get_sparse_core_info()
NC = _info.num_cores
NS = _info.num_subcores
L = _info.num_lanes
NW = NC * NS

E_PER = 49920
CHUNK = 9984
NCHUNK = E_PER // CHUNK
VECS = CHUNK // L
TAIL_OFF = E_PER * 32
TAIL = E - TAIL_OFF

NBUF = 2
U = 6

assert NCHUNK * CHUNK == E_PER and VECS * L == CHUNK
assert VECS % U == 0 and E_PER % 128 == 0 and CHUNK % 128 == 0

_mesh = plsc.VectorSubcoreMesh(core_axis_name="c", subcore_axis_name="s")


@functools.partial(
    pl.kernel,
    mesh=_mesh,
    compiler_params=pltpu.CompilerParams(needs_layout_passes=False),
    out_type=jax.ShapeDtypeStruct(((NW + 1) * NOUT,), jnp.float32),
    scratch_types=[
        pltpu.VMEM((NIN,), jnp.float32),
        pltpu.VMEM((NOUT,), jnp.float32),
        pltpu.VMEM((2, CHUNK), jnp.int32),
        pltpu.VMEM((2, CHUNK), jnp.int32),
        pltpu.VMEM((CHUNK,), jnp.float32),
        pltpu.VMEM((CHUNK,), jnp.float32),
        pltpu.VMEM((2, TAIL), jnp.int32),
        pltpu.VMEM((TAIL,), jnp.float32),
        pltpu.SemaphoreType.DMA,
        pltpu.SemaphoreType.DMA,
    ],
)
def _edge_scan(x_hbm, edge_hbm, w_hbm, b_hbm, out_hbm,
               x_v, acc_v, ed0, ed1, w0, w1, edt, wt, sem0, sem1):
    sems = (sem0, sem1)
    eds = (ed0, ed1)
    ws = (w0, w1)
    wid = lax.axis_index("s") * NC + lax.axis_index("c")
    base = wid * E_PER

    pltpu.sync_copy(x_hbm, x_v)

    zero = jnp.zeros((L,), jnp.float32)

    def zbody(i, carry):
        acc_v[pl.ds(i * L, L)] = zero
        return carry

    lax.fori_loop(0, NOUT // L, zbody, 0)

    def start(c):
        b = c % NBUF
        off = base + c * CHUNK
        return [
            pltpu.async_copy(edge_hbm.at[:, pl.ds(off, CHUNK)], eds[b], sems[b]),
            pltpu.async_copy(w_hbm.at[pl.ds(off, CHUNK)], ws[b], sems[b]),
        ]

    handles = start(0)
    for c in range(NCHUNK):
        nxt = start(c + 1) if c + 1 < NCHUNK else None
        for h in handles:
            h.wait()
        handles = nxt
        b = c % NBUF
        ev, wv = eds[b], ws[b]

        @plsc.parallel_loop(0, CHUNK, L, unroll=U)
        def _(o):
            s = ev[0, pl.ds(o, L)]
            d = ev[1, pl.ds(o, L)]
            w = wv[pl.ds(o, L)]
            m = (s < NIN) & (d >= LO)
            xg = plsc.load_gather(x_v, [s & (NIN - 1)])
            plsc.addupdate_scatter(
                acc_v, [(d - LO) & (NOUT - 1)], w * xg, mask=m)

    @pl.when(wid == 0)
    def _():
        pltpu.sync_copy(edge_hbm.at[:, pl.ds(TAIL_OFF, TAIL)], edt)
        pltpu.sync_copy(w_hbm.at[pl.ds(TAIL_OFF, TAIL)], wt)

        @plsc.parallel_loop(0, TAIL, L, unroll=5)
        def _(o):
            s = edt[0, pl.ds(o, L)]
            d = edt[1, pl.ds(o, L)]
            w = wt[pl.ds(o, L)]
            m = (s < NIN) & (d >= LO)
            xg = plsc.load_gather(x_v, [s & (NIN - 1)])
            plsc.addupdate_scatter(
                acc_v, [(d - LO) & (NOUT - 1)], w * xg, mask=m)

        pltpu.sync_copy(b_hbm.at[pl.ds(LO, NOUT)], x_v)
        pltpu.sync_copy(x_v, out_hbm.at[pl.ds(NW * NOUT, NOUT)])

    pltpu.sync_copy(acc_v, out_hbm.at[pl.ds(wid * NOUT, NOUT)])


def _combine_body(p_ref, o_ref):
    o_ref[...] = jnp.tanh(jnp.sum(p_ref[...], axis=0, keepdims=True))


def kernel(x, edge_index, weights, biases):
    partials = _edge_scan(x, edge_index, weights, biases)
    out = pl.pallas_call(
        _combine_body,
        out_shape=jax.ShapeDtypeStruct((1, NOUT), jnp.float32),
    )(partials.reshape(NW + 1, NOUT))
    return out.reshape(NOUT)
